# baseline (speedup 1.0000x reference)
import functools

import jax
import jax.numpy as jnp
from jax import lax
from jax.experimental import pallas as pl
from jax.experimental.pallas import tpu as pltpu
from jax.experimental.pallas import tpu_sc as plsc

N = 4096
C = 16
M = 65536
NE = 4
NY = 10
H = 128

NCORE = 2
NSUB = 16
NWORK = NCORE * NSUB
PAIRS = 2 * M
PER_TEC = PAIRS // NWORK
CHUNK = 128
N_CHUNKS = PER_TEC // CHUNK
ROWS_PER_TEC = N // NSUB


_KX = (0x6D3E048F, 0x1022172D)
_KE = (0x03D7B32D, 0xADD083F4)


def _tf_uniform(key, flat_idx):
    k0 = jnp.uint32(key[0])
    k1 = jnp.uint32(key[1])
    ks = (k0, k1, k0 ^ k1 ^ jnp.uint32(0x1BD11BDA))
    rot = ((13, 15, 26, 6), (17, 29, 16, 24))
    x1 = flat_idx.astype(jnp.uint32)
    x0 = jnp.zeros_like(x1) + ks[0]
    x1 = x1 + ks[1]
    for i in range(5):
        for r in rot[i % 2]:
            x0 = x0 + x1
            x1 = (x1 << jnp.uint32(r)) | (x1 >> jnp.uint32(32 - r))
            x1 = x1 ^ x0
        x0 = x0 + ks[(i + 1) % 3]
        x1 = x1 + ks[(i + 2) % 3] + jnp.uint32(i + 1)
    bits = x0 ^ x1
    fl = lax.bitcast_convert_type(
        (bits >> jnp.uint32(9)) | jnp.uint32(0x3F800000), jnp.float32) - 1.0
    tiny = jnp.finfo(jnp.float32).tiny
    return jnp.maximum(tiny, fl * (1.0 - tiny) + tiny)


def _tc_sample_body(lx_ref, let_ref, src_ref, dst_ref, w1_ref,
                    h_ref, g_ref, s_ref):
    lx = lx_ref[...]
    ii = lax.broadcasted_iota(jnp.int32, (N, C), 1)
    ir = lax.broadcasted_iota(jnp.int32, (N, C), 0)
    ux = _tf_uniform(_KX, ir * C + ii)
    gx = -jnp.log(-jnp.log(ux))
    sx = lx + gx
    amax = jnp.max(sx, axis=1, keepdims=True)
    idxm = jnp.min(jnp.where(sx >= amax, ii, C), axis=1, keepdims=True)
    oh = (ii == idxm).astype(jnp.float32)
    h_ref[...] = jnp.dot(oh, w1_ref[...], preferred_element_type=jnp.float32)

    er = lax.broadcasted_iota(jnp.int32, (M // 128, 128), 0)
    el = lax.broadcasted_iota(jnp.int32, (M // 128, 128), 1)
    jj4 = (er * 128 + el) * NE
    sc = []
    for c in range(NE):
        uc = _tf_uniform(_KE, jj4 + c)
        sc.append(let_ref[c] - jnp.log(-jnp.log(uc)))
    s0, s1, s2, s3 = sc
    m = (s1 > s0) | (s2 > s0) | (s3 > s0)
    src = src_ref[...]
    dst = dst_ref[...]
    trash = jnp.full(src.shape, N, dtype=jnp.int32)
    g_ref[0] = dst
    g_ref[1] = src
    s_ref[0] = jnp.where(m, src, trash)
    s_ref[1] = jnp.where(m, dst, trash)


_tc_sample = pl.pallas_call(
    _tc_sample_body,
    out_shape=[
        jax.ShapeDtypeStruct((N, H), jnp.float32),
        jax.ShapeDtypeStruct((2, M // 128, 128), jnp.int32),
        jax.ShapeDtypeStruct((2, M // 128, 128), jnp.int32),
    ],
)


_NBUF = 4


def _sc_agg_body(h_hbm, g_hbm, s_hbm, out_hbm, g_all, s_all, rows, gsems, ssems,
                 agg_sh):
    cid = lax.axis_index("c")
    sid = lax.axis_index("s")
    wid = cid * NSUB + sid
    blk = wid * N_CHUNKS

    pltpu.sync_copy(g_hbm.at[pl.ds(blk, N_CHUNKS)], g_all)
    pltpu.sync_copy(s_hbm.at[pl.ds(blk, N_CHUNKS)], s_all)
    for k in range(_NBUF):
        pltpu.async_copy(h_hbm.at[g_all.at[k]], rows[k], gsems[k])

    for hop in range(ROWS_PER_TEC // CHUNK):
        r0 = sid * ROWS_PER_TEC + hop * CHUNK
        pltpu.sync_copy(h_hbm.at[pl.ds(r0, CHUNK)], agg_sh.at[pl.ds(r0, CHUNK)])
    plsc.subcore_barrier()

    def step(i, carry):
        scat = []
        for k in range(_NBUF):
            j = _NBUF * i + k
            pltpu.make_async_copy(
                h_hbm.at[pl.ds(0, CHUNK)], rows[k], gsems[k]).wait()
            scat.append(pltpu.async_copy(
                rows[k], agg_sh.at[s_all.at[j]], ssems[k], add=True))
        for k in range(_NBUF):
            scat[k].wait()

            @pl.when(i < N_CHUNKS // _NBUF - 1)
            def _():
                pltpu.async_copy(
                    h_hbm.at[g_all.at[_NBUF * (i + 1) + k]], rows[k], gsems[k])
        return carry

    lax.fori_loop(0, N_CHUNKS // _NBUF, step, 0)
    plsc.subcore_barrier()

    for hop in range(ROWS_PER_TEC // CHUNK):
        r0 = sid * ROWS_PER_TEC + hop * CHUNK
        pltpu.sync_copy(agg_sh.at[pl.ds(r0, CHUNK)], rows[0])
        pltpu.sync_copy(rows[0], out_hbm.at[cid, pl.ds(r0, CHUNK)])


@functools.lru_cache(maxsize=None)
def _get_sc_agg():
    return pl.kernel(
        _sc_agg_body,
        out_type=jax.ShapeDtypeStruct((NCORE, N, H), jnp.float32),
        mesh=plsc.VectorSubcoreMesh(core_axis_name="c", subcore_axis_name="s",
                                    num_cores=NCORE, num_subcores=NSUB),
        scratch_types=[
            pltpu.VMEM((N_CHUNKS, CHUNK), jnp.int32),
            pltpu.VMEM((N_CHUNKS, CHUNK), jnp.int32),
            [pltpu.VMEM((CHUNK, H), jnp.float32) for _ in range(_NBUF)],
            [pltpu.SemaphoreType.DMA for _ in range(_NBUF)],
            [pltpu.SemaphoreType.DMA for _ in range(_NBUF)],
            pltpu.VMEM_SHARED((N + 8, H), jnp.float32),
        ],
    )


def _tc_head_body(a_ref, h_ref, w2_ref, y_ref, loss_ref):
    z = jnp.maximum(a_ref[0] + a_ref[1] - h_ref[...], 0.0)
    ly = jnp.dot(z, w2_ref[...], preferred_element_type=jnp.float32)
    mx = jnp.max(ly, axis=1, keepdims=True)
    lse = jnp.log(jnp.sum(jnp.exp(ly - mx), axis=1, keepdims=True)) + mx
    logp = ly - lse
    ohy = (lax.broadcasted_iota(jnp.int32, (N, NY), 1) == y_ref[...])
    loss_ref[...] = jnp.reshape(
        -jnp.sum(jnp.where(ohy, logp, 0.0)) / N, (1, 1))


_tc_head = pl.pallas_call(
    _tc_head_body,
    out_shape=jax.ShapeDtypeStruct((1, 1), jnp.float32),
)


def kernel(logit_X, logit_E, Y, src, dst, W1, W2):
    leT = logit_E.T.reshape(NE, M // 128, 128)
    srcR = src.astype(jnp.int32).reshape(M // 128, 128)
    dstR = dst.astype(jnp.int32).reshape(M // 128, 128)
    h, g, s = _tc_sample(logit_X, leT, srcR, dstR, W1)
    a = _get_sc_agg()(h, g.reshape(PAIRS // CHUNK, CHUNK),
                      s.reshape(PAIRS // CHUNK, CHUNK))
    loss = _tc_head(a, h, W2, Y.astype(jnp.int32).reshape(N, 1))
    return loss.reshape(())


# speedup vs baseline: 1.0059x; 1.0059x over previous
import functools

import jax
import jax.numpy as jnp
from jax import lax
from jax.experimental import pallas as pl
from jax.experimental.pallas import tpu as pltpu
from jax.experimental.pallas import tpu_sc as plsc

N = 4096
C = 16
M = 65536
NE = 4
NY = 10
H = 128

NCORE = 2
NSUB = 16
NWORK = NCORE * NSUB
PAIRS = 2 * M
PER_TEC = PAIRS // NWORK
CHUNK = 128
N_CHUNKS = PER_TEC // CHUNK
ROWS_PER_TEC = N // NSUB


_KX = (0x6D3E048F, 0x1022172D)
_KE = (0x03D7B32D, 0xADD083F4)


def _tf_uniform(key, flat_idx):
    k0 = jnp.uint32(key[0])
    k1 = jnp.uint32(key[1])
    ks = (k0, k1, k0 ^ k1 ^ jnp.uint32(0x1BD11BDA))
    rot = ((13, 15, 26, 6), (17, 29, 16, 24))
    x1 = flat_idx.astype(jnp.uint32)
    x0 = jnp.zeros_like(x1) + ks[0]
    x1 = x1 + ks[1]
    for i in range(5):
        for r in rot[i % 2]:
            x0 = x0 + x1
            x1 = (x1 << jnp.uint32(r)) | (x1 >> jnp.uint32(32 - r))
            x1 = x1 ^ x0
        x0 = x0 + ks[(i + 1) % 3]
        x1 = x1 + ks[(i + 2) % 3] + jnp.uint32(i + 1)
    bits = x0 ^ x1
    fl = lax.bitcast_convert_type(
        (bits >> jnp.uint32(9)) | jnp.uint32(0x3F800000), jnp.float32) - 1.0
    tiny = jnp.finfo(jnp.float32).tiny
    return jnp.maximum(tiny, fl * (1.0 - tiny) + tiny)


def _tc_sample_body(lx_ref, let_ref, src_ref, dst_ref, w1_ref,
                    h_ref, g_ref, s_ref):
    lx = lx_ref[...]
    ii = lax.broadcasted_iota(jnp.int32, (N, C), 1)
    ir = lax.broadcasted_iota(jnp.int32, (N, C), 0)
    ux = _tf_uniform(_KX, ir * C + ii)
    gx = -jnp.log(-jnp.log(ux))
    sx = lx + gx
    amax = jnp.max(sx, axis=1, keepdims=True)
    idxm = jnp.min(jnp.where(sx >= amax, ii, C), axis=1, keepdims=True)
    oh = (ii == idxm).astype(jnp.float32)
    h_ref[...] = jnp.dot(oh, w1_ref[...], preferred_element_type=jnp.float32)

    er = lax.broadcasted_iota(jnp.int32, (M // 128, 128), 0)
    el = lax.broadcasted_iota(jnp.int32, (M // 128, 128), 1)
    jj4 = (er * 128 + el) * NE
    sc = []
    for c in range(NE):
        uc = _tf_uniform(_KE, jj4 + c)
        sc.append(let_ref[c] - jnp.log(-jnp.log(uc)))
    s0, s1, s2, s3 = sc
    m = (s1 > s0) | (s2 > s0) | (s3 > s0)
    src = src_ref[...]
    dst = dst_ref[...]
    trash = jnp.full(src.shape, N, dtype=jnp.int32)
    g_ref[0] = dst
    g_ref[1] = src
    s_ref[0] = jnp.where(m, src, trash)
    s_ref[1] = jnp.where(m, dst, trash)


_tc_sample = pl.pallas_call(
    _tc_sample_body,
    out_shape=[
        jax.ShapeDtypeStruct((N, H), jnp.float32),
        jax.ShapeDtypeStruct((2, M // 128, 128), jnp.int32),
        jax.ShapeDtypeStruct((2, M // 128, 128), jnp.int32),
    ],
)


_NBUF = 4


def _sc_agg_body(h_hbm, g_hbm, s_hbm, out_hbm, g_all, s_all, rows, gsems, ssems,
                 agg_sh):
    cid = lax.axis_index("c")
    sid = lax.axis_index("s")
    wid = cid * NSUB + sid
    blk = wid * N_CHUNKS

    pltpu.sync_copy(g_hbm.at[pl.ds(blk, N_CHUNKS)], g_all)
    pltpu.sync_copy(s_hbm.at[pl.ds(blk, N_CHUNKS)], s_all)
    for k in range(_NBUF):
        pltpu.async_copy(h_hbm.at[g_all.at[k]], rows[k], gsems[k])

    for hop in range(ROWS_PER_TEC // CHUNK):
        r0 = sid * ROWS_PER_TEC + hop * CHUNK
        pltpu.sync_copy(h_hbm.at[pl.ds(r0, CHUNK)], agg_sh.at[pl.ds(r0, CHUNK)])
    plsc.subcore_barrier()

    def step(i, carry):
        scat = []
        for k in range(_NBUF):
            j = _NBUF * i + k
            pltpu.make_async_copy(
                h_hbm.at[pl.ds(0, CHUNK)], rows[k], gsems[k]).wait()
            scat.append(pltpu.async_copy(
                rows[k], agg_sh.at[s_all.at[j]], ssems[k], add=True))
        for k in range(_NBUF):
            scat[k].wait()

            @pl.when(i < N_CHUNKS // _NBUF - 1)
            def _():
                pltpu.async_copy(
                    h_hbm.at[g_all.at[_NBUF * (i + 1) + k]], rows[k], gsems[k])
        return carry

    lax.fori_loop(0, N_CHUNKS // _NBUF, step, 0)
    plsc.subcore_barrier()

    for hop in range(ROWS_PER_TEC // CHUNK):
        r0 = sid * ROWS_PER_TEC + hop * CHUNK
        pltpu.sync_copy(agg_sh.at[pl.ds(r0, CHUNK)], rows[0])
        pltpu.sync_copy(rows[0], out_hbm.at[cid, pl.ds(r0, CHUNK)])


@functools.lru_cache(maxsize=None)
def _get_sc_agg():
    return pl.kernel(
        _sc_agg_body,
        out_type=jax.ShapeDtypeStruct((NCORE, N, H), jnp.float32),
        mesh=plsc.VectorSubcoreMesh(core_axis_name="c", subcore_axis_name="s",
                                    num_cores=NCORE, num_subcores=NSUB),
        scratch_types=[
            pltpu.VMEM((N_CHUNKS, CHUNK), jnp.int32),
            pltpu.VMEM((N_CHUNKS, CHUNK), jnp.int32),
            [pltpu.VMEM((CHUNK, H), jnp.float32) for _ in range(_NBUF)],
            [pltpu.SemaphoreType.DMA for _ in range(_NBUF)],
            [pltpu.SemaphoreType.DMA for _ in range(_NBUF)],
            pltpu.VMEM_SHARED((N + 8, H), jnp.float32),
        ],
    )


def _tc_head_body(a_ref, h_ref, w2_ref, ohy_ref, loss_ref):
    z = jnp.maximum(a_ref[0] + a_ref[1] - h_ref[...], 0.0)
    ly = jnp.dot(z, w2_ref[...], preferred_element_type=jnp.float32)
    mx = jnp.max(ly, axis=1, keepdims=True)
    lse = jnp.log(jnp.sum(jnp.exp(ly - mx), axis=1, keepdims=True)) + mx
    logp = ly - lse
    loss_ref[...] = jnp.reshape(-jnp.sum(logp * ohy_ref[...]) / N, (1, 1))


_tc_head = pl.pallas_call(
    _tc_head_body,
    out_shape=jax.ShapeDtypeStruct((1, 1), jnp.float32),
)


def kernel(logit_X, logit_E, Y, src, dst, W1, W2):
    leT = logit_E.T.reshape(NE, M // 128, 128)
    srcR = src.astype(jnp.int32).reshape(M // 128, 128)
    dstR = dst.astype(jnp.int32).reshape(M // 128, 128)
    h, g, s = _tc_sample(logit_X, leT, srcR, dstR, W1)
    a = _get_sc_agg()(h, g.reshape(PAIRS // CHUNK, CHUNK),
                      s.reshape(PAIRS // CHUNK, CHUNK))
    ohY = jax.nn.one_hot(Y, NY, dtype=jnp.float32)
    loss = _tc_head(a, h, W2, ohY)
    return loss.reshape(())
